# baseline (speedup 1.0000x reference)
import functools
import math

import jax
import jax.numpy as jnp
from jax import lax
from jax.experimental import pallas as pl
from jax.experimental.pallas import tpu as pltpu
from jax.experimental.pallas import tpu_sc as plsc

VOCAB = 100000
EMBED_DIM = 128
SEQ_LEN = 200
BATCH = 1024

L = 16
NW = 32
ROWS_PER_W = (BATCH * SEQ_LEN) // NW
CHUNK = 128
NCHUNK = ROWS_PER_W // CHUNK
PE_ROWS = 320
NBUF = 4


def _sc_kernel(table_hbm, idx_hbm, pe_hbm, out_hbm, idx_v, pe_v,
               buf0, buf1, buf2, buf3, g0, g1, g2, g3, o0, o1, o2, o3):
    bufs = (buf0, buf1, buf2, buf3)
    gsems = (g0, g1, g2, g3)
    osems = (o0, o1, o2, o3)
    nc = 2
    wid = lax.axis_index("s") * nc + lax.axis_index("c")
    base = wid * ROWS_PER_W
    pltpu.sync_copy(idx_hbm.at[pl.ds(base, ROWS_PER_W)], idx_v)
    pltpu.sync_copy(pe_hbm, pe_v)

    def gather_desc(c, b):
        off = pl.multiple_of(c * CHUNK, 8)
        return pltpu.make_async_copy(
            table_hbm.at[idx_v.at[pl.ds(off, CHUNK)]], bufs[b], gsems[b])

    def write_desc(c, b):
        off = pl.multiple_of(c * CHUNK, 8)
        return pltpu.make_async_copy(
            bufs[b], out_hbm.at[pl.ds(base + off, CHUNK)], osems[b])

    def add_pe(c, b):
        p0 = lax.rem(c * CHUNK, SEQ_LEN)
        buf = bufs[b]

        @plsc.parallel_loop(0, CHUNK, 1, unroll=4)
        def _row(r):
            for j in range(EMBED_DIM // L):
                sl = pl.ds(j * L, L)
                buf[r, sl] = buf[r, sl] + pe_v[p0 + r, sl]

    gather_desc(0, 0).start()
    gather_desc(1, 1).start()
    for v in (0, 1):
        gather_desc(v + 2, v + 2).start()
        gather_desc(v, v).wait()
        add_pe(v, v)
        write_desc(v, v).start()

    def steady(v, b, tb):
        write_desc(v - 2, tb).wait()
        gather_desc(v + 2, tb).start()
        gather_desc(v, b).wait()
        add_pe(v, b)
        write_desc(v, b).start()

    ngroups = (NCHUNK - 4) // 4

    def group(g, _):
        v0 = 2 + g * 4
        for i in range(4):
            steady(v0 + i, (2 + i) % 4, i)
        return ()

    lax.fori_loop(0, ngroups, group, ())

    for v in range(2 + ngroups * 4, NCHUNK - 2):
        steady(v, v % 4, (v + 2) % 4)

    for v in (NCHUNK - 2, NCHUNK - 1):
        b = v % 4
        gather_desc(v, b).wait()
        add_pe(v, b)
        write_desc(v, b).start()
    for v in range(NCHUNK - 4, NCHUNK):
        write_desc(v, v % 4).wait()


@jax.jit
def kernel(token_ids, table, positional_encoding):
    ids = token_ids.astype(jnp.int32).reshape(-1)
    pe = positional_encoding[0, :SEQ_LEN].astype(jnp.float32)
    pe2 = jnp.concatenate([pe, pe[: PE_ROWS - SEQ_LEN]], axis=0)

    mesh = plsc.VectorSubcoreMesh(core_axis_name="c", subcore_axis_name="s")
    out = pl.kernel(
        _sc_kernel,
        mesh=mesh,
        out_type=jax.ShapeDtypeStruct((BATCH * SEQ_LEN, EMBED_DIM), jnp.float32),
        scratch_types=[
            pltpu.VMEM((ROWS_PER_W,), jnp.int32),
            pltpu.VMEM((PE_ROWS, EMBED_DIM), jnp.float32),
        ] + [pltpu.VMEM((CHUNK, EMBED_DIM), jnp.float32)] * NBUF
          + [pltpu.SemaphoreType.DMA] * (2 * NBUF),
    )(table, ids, pe2)
    return out.reshape(BATCH, SEQ_LEN, EMBED_DIM)


# speedup vs baseline: 1.0098x; 1.0098x over previous
import functools
import math

import jax
import jax.numpy as jnp
from jax import lax
from jax.experimental import pallas as pl
from jax.experimental.pallas import tpu as pltpu
from jax.experimental.pallas import tpu_sc as plsc

VOCAB = 100000
EMBED_DIM = 128
SEQ_LEN = 200
BATCH = 1024

L = 16
NW = 32
ROWS_PER_W = (BATCH * SEQ_LEN) // NW
CHUNK = 80
NCHUNK = ROWS_PER_W // CHUNK
PE_ROWS = 240
NBUF = 4


def _sc_kernel(table_hbm, idx_hbm, pe_hbm, out_hbm, idx_v, pe_v,
               buf0, buf1, buf2, buf3, g0, g1, g2, g3, o0, o1, o2, o3):
    bufs = (buf0, buf1, buf2, buf3)
    gsems = (g0, g1, g2, g3)
    osems = (o0, o1, o2, o3)
    nc = 2
    wid = lax.axis_index("s") * nc + lax.axis_index("c")
    base = wid * ROWS_PER_W
    pltpu.sync_copy(idx_hbm.at[pl.ds(base, ROWS_PER_W)], idx_v)
    pltpu.sync_copy(pe_hbm, pe_v)

    def gather_desc(c, b):
        off = pl.multiple_of(c * CHUNK, 8)
        return pltpu.make_async_copy(
            table_hbm.at[idx_v.at[pl.ds(off, CHUNK)]], bufs[b], gsems[b])

    def write_desc(c, b):
        off = pl.multiple_of(c * CHUNK, 8)
        return pltpu.make_async_copy(
            bufs[b], out_hbm.at[pl.ds(base + off, CHUNK)], osems[b])

    def add_pe(c, b):
        p0 = lax.rem(c * CHUNK, SEQ_LEN)
        buf = bufs[b]

        @plsc.parallel_loop(0, CHUNK, 1, unroll=4)
        def _row(r):
            for j in range(EMBED_DIM // L):
                sl = pl.ds(j * L, L)
                buf[r, sl] = buf[r, sl] + pe_v[p0 + r, sl]

    gather_desc(0, 0).start()
    gather_desc(1, 1).start()
    for v in (0, 1):
        gather_desc(v + 2, v + 2).start()
        gather_desc(v, v).wait()
        add_pe(v, v)
        write_desc(v, v).start()

    def group(g, _):
        v0 = 2 + g * 4
        for i in range(4):
            v = v0 + i
            b = (2 + i) % 4
            tb = i
            write_desc(v - 2, tb).wait()
            gather_desc(v + 2, tb).start()
            gather_desc(v, b).wait()
            add_pe(v, b)
            write_desc(v, b).start()
        return ()

    lax.fori_loop(0, (NCHUNK - 4) // 4, group, ())

    for v in (NCHUNK - 2, NCHUNK - 1):
        b = v % 4
        gather_desc(v, b).wait()
        add_pe(v, b)
        write_desc(v, b).start()
    for b in range(4):
        write_desc(NCHUNK - 4 + b, b).wait()


@jax.jit
def kernel(token_ids, table, positional_encoding):
    ids = token_ids.astype(jnp.int32).reshape(-1)
    pe = positional_encoding[0, :SEQ_LEN].astype(jnp.float32)
    pe2 = jnp.concatenate([pe, pe[: PE_ROWS - SEQ_LEN]], axis=0)

    mesh = plsc.VectorSubcoreMesh(core_axis_name="c", subcore_axis_name="s")
    out = pl.kernel(
        _sc_kernel,
        mesh=mesh,
        out_type=jax.ShapeDtypeStruct((BATCH * SEQ_LEN, EMBED_DIM), jnp.float32),
        scratch_types=[
            pltpu.VMEM((ROWS_PER_W,), jnp.int32),
            pltpu.VMEM((PE_ROWS, EMBED_DIM), jnp.float32),
        ] + [pltpu.VMEM((CHUNK, EMBED_DIM), jnp.float32)] * NBUF
          + [pltpu.SemaphoreType.DMA] * (2 * NBUF),
    )(table, ids, pe2)
    return out.reshape(BATCH, SEQ_LEN, EMBED_DIM)


# speedup vs baseline: 1.0234x; 1.0135x over previous
import functools
import math

import jax
import jax.numpy as jnp
from jax import lax
from jax.experimental import pallas as pl
from jax.experimental.pallas import tpu as pltpu
from jax.experimental.pallas import tpu_sc as plsc

VOCAB = 100000
EMBED_DIM = 128
SEQ_LEN = 200
BATCH = 1024

L = 16
NW = 32
ROWS_PER_W = (BATCH * SEQ_LEN) // NW
CHUNK = 200
NCHUNK = ROWS_PER_W // CHUNK
NBUF = 3


def _sc_kernel(table_hbm, idx_hbm, pe_hbm, out_hbm, idx_v, pe_v,
               buf0, buf1, buf2, g0, g1, g2, o0, o1, o2):
    bufs = (buf0, buf1, buf2)
    gsems = (g0, g1, g2)
    osems = (o0, o1, o2)
    nc = 2
    wid = lax.axis_index("s") * nc + lax.axis_index("c")
    base = wid * ROWS_PER_W
    pltpu.sync_copy(idx_hbm.at[pl.ds(base, ROWS_PER_W)], idx_v)
    pltpu.sync_copy(pe_hbm, pe_v)

    def gather_desc(c, b):
        off = pl.multiple_of(c * CHUNK, 8)
        return pltpu.make_async_copy(
            table_hbm.at[idx_v.at[pl.ds(off, CHUNK)]], bufs[b], gsems[b])

    def write_desc(c, b):
        off = pl.multiple_of(c * CHUNK, 8)
        return pltpu.make_async_copy(
            bufs[b], out_hbm.at[pl.ds(base + off, CHUNK)], osems[b])

    def add_pe(b):
        buf = bufs[b]

        @plsc.parallel_loop(0, CHUNK, 1, unroll=4)
        def _row(r):
            for j in range(EMBED_DIM // L):
                sl = pl.ds(j * L, L)
                buf[r, sl] = buf[r, sl] + pe_v[r, sl]

    gather_desc(0, 0).start()
    gather_desc(1, 1).start()
    gather_desc(0, 0).wait()
    add_pe(0)
    write_desc(0, 0).start()
    gather_desc(2, 2).start()
    gather_desc(1, 1).wait()
    add_pe(1)
    write_desc(1, 1).start()

    def group(g, _):
        v0 = 2 + g * 3
        for i in range(3):
            v = v0 + i
            b = (2 + i) % 3
            tb = i
            write_desc(v - 2, tb).wait()
            gather_desc(v + 1, tb).start()
            gather_desc(v, b).wait()
            add_pe(b)
            write_desc(v, b).start()
        return ()

    lax.fori_loop(0, (NCHUNK - 5) // 3, group, ())

    for v in (NCHUNK - 3, NCHUNK - 2):
        b = v % 3
        tb = (v + 1) % 3
        write_desc(v - 2, tb).wait()
        gather_desc(v + 1, tb).start()
        gather_desc(v, b).wait()
        add_pe(b)
        write_desc(v, b).start()
    v = NCHUNK - 1
    b = v % 3
    gather_desc(v, b).wait()
    add_pe(b)
    write_desc(v, b).start()
    for v in (NCHUNK - 3, NCHUNK - 2, NCHUNK - 1):
        write_desc(v, v % 3).wait()


@jax.jit
def kernel(token_ids, table, positional_encoding):
    ids = token_ids.astype(jnp.int32).reshape(-1)
    pe = positional_encoding[0, :SEQ_LEN].astype(jnp.float32)

    mesh = plsc.VectorSubcoreMesh(core_axis_name="c", subcore_axis_name="s")
    out = pl.kernel(
        _sc_kernel,
        mesh=mesh,
        out_type=jax.ShapeDtypeStruct((BATCH * SEQ_LEN, EMBED_DIM), jnp.float32),
        scratch_types=[
            pltpu.VMEM((ROWS_PER_W,), jnp.int32),
            pltpu.VMEM((SEQ_LEN, EMBED_DIM), jnp.float32),
        ] + [pltpu.VMEM((CHUNK, EMBED_DIM), jnp.float32)] * NBUF
          + [pltpu.SemaphoreType.DMA] * (2 * NBUF),
    )(table, ids, pe)
    return out.reshape(BATCH, SEQ_LEN, EMBED_DIM)


# speedup vs baseline: 1.0492x; 1.0252x over previous
import functools
import math

import jax
import jax.numpy as jnp
from jax import lax
from jax.experimental import pallas as pl
from jax.experimental.pallas import tpu as pltpu
from jax.experimental.pallas import tpu_sc as plsc

VOCAB = 100000
EMBED_DIM = 128
SEQ_LEN = 200
BATCH = 1024

L = 16
NW = 32
ROWS_PER_W = (BATCH * SEQ_LEN) // NW
CHUNK = 200
NCHUNK = ROWS_PER_W // CHUNK
NBUF = 3


def _sc_kernel(table_hbm, idx_hbm, pe_hbm, out_hbm, idx_v, pe_v,
               buf0, buf1, buf2, g0, g1, g2, o0, o1, o2):
    bufs = (buf0, buf1, buf2)
    gsems = (g0, g1, g2)
    osems = (o0, o1, o2)
    nc = 2
    wid = lax.axis_index("s") * nc + lax.axis_index("c")
    base = wid * ROWS_PER_W
    pltpu.sync_copy(idx_hbm.at[pl.ds(base, ROWS_PER_W)], idx_v)
    pltpu.sync_copy(pe_hbm, pe_v)

    def gather_desc(c, b):
        off = pl.multiple_of(c * CHUNK, 8)
        return pltpu.make_async_copy(
            table_hbm.at[idx_v.at[pl.ds(off, CHUNK)]], bufs[b], gsems[b])

    def write_desc(c, b):
        off = pl.multiple_of(c * CHUNK, 8)
        return pltpu.make_async_copy(
            bufs[b], out_hbm.at[pl.ds(base + off, CHUNK)], osems[b])

    def fill_pe(b):
        buf = bufs[b]

        @plsc.parallel_loop(0, CHUNK, 1, unroll=4)
        def _row(r):
            for j in range(EMBED_DIM // L):
                sl = pl.ds(j * L, L)
                buf[r, sl] = pe_v[r, sl]

    for b in range(3):
        fill_pe(b)
        gather_desc(b, b).start(add=True)
    for v in (0, 1):
        gather_desc(v, v).wait()
        write_desc(v, v).start()

    def group(g, _):
        v0 = 2 + g * 3
        for i in range(3):
            v = v0 + i
            b = (2 + i) % 3
            tb = i
            write_desc(v - 2, tb).wait()
            fill_pe(tb)
            gather_desc(v + 1, tb).start(add=True)
            gather_desc(v, b).wait()
            write_desc(v, b).start()
        return ()

    lax.fori_loop(0, (NCHUNK - 5) // 3, group, ())

    for v in (NCHUNK - 3, NCHUNK - 2):
        b = v % 3
        tb = (v + 1) % 3
        write_desc(v - 2, tb).wait()
        fill_pe(tb)
        gather_desc(v + 1, tb).start(add=True)
        gather_desc(v, b).wait()
        write_desc(v, b).start()
    v = NCHUNK - 1
    b = v % 3
    gather_desc(v, b).wait()
    write_desc(v, b).start()
    for v in (NCHUNK - 3, NCHUNK - 2, NCHUNK - 1):
        write_desc(v, v % 3).wait()


@jax.jit
def kernel(token_ids, table, positional_encoding):
    ids = token_ids.astype(jnp.int32).reshape(-1)
    pe = positional_encoding[0, :SEQ_LEN].astype(jnp.float32)

    mesh = plsc.VectorSubcoreMesh(core_axis_name="c", subcore_axis_name="s")
    out = pl.kernel(
        _sc_kernel,
        mesh=mesh,
        out_type=jax.ShapeDtypeStruct((BATCH * SEQ_LEN, EMBED_DIM), jnp.float32),
        scratch_types=[
            pltpu.VMEM((ROWS_PER_W,), jnp.int32),
            pltpu.VMEM((SEQ_LEN, EMBED_DIM), jnp.float32),
        ] + [pltpu.VMEM((CHUNK, EMBED_DIM), jnp.float32)] * NBUF
          + [pltpu.SemaphoreType.DMA] * (2 * NBUF),
    )(table, ids, pe)
    return out.reshape(BATCH, SEQ_LEN, EMBED_DIM)
